# baseline (speedup 1.0000x reference)
import functools

import jax
import jax.numpy as jnp
from jax.experimental import pallas as pl
from jax.experimental.pallas import tpu as pltpu

_EPS = 1e-6
_CAPACITY = 0.5


def _swiglu_chunk(xn, wg_ref, wu_ref, wd_ref):
    a = jnp.dot(xn, wg_ref[...], preferred_element_type=jnp.float32)
    b = jnp.dot(xn, wu_ref[...], preferred_element_type=jnp.float32)
    h = (a * jax.nn.sigmoid(a)) * b
    return jnp.dot(h, wd_ref[...], preferred_element_type=jnp.float32)


def _row_threshold(gb, k, t):
    def body1(_, lohi):
        lo, hi = lohi
        mid = lo + (hi - lo + 1) // 2
        cnt = jnp.sum((gb >= mid).astype(jnp.int32))
        pred = cnt >= k
        return jnp.where(pred, mid, lo), jnp.where(pred, hi, mid - 1)

    z = jnp.zeros((), jnp.int32)
    thresh, _ = jax.lax.fori_loop(
        0, 31, body1, (z, jnp.full((), 0x3F800000, jnp.int32)))

    n_gt = jnp.sum((gb > thresh).astype(jnp.int32))
    need = k - n_gt
    eq = gb == thresh
    lanes = gb.shape[-1]
    iota = (jax.lax.broadcasted_iota(jnp.int32, gb.shape, 0) * lanes
            + jax.lax.broadcasted_iota(jnp.int32, gb.shape, 1))

    def body2(_, lohi):
        lo2, hi2 = lohi
        mid = (lo2 + hi2) // 2
        q = jnp.sum((eq & (iota < mid)).astype(jnp.int32)) >= need
        return jnp.where(q, lo2, mid), jnp.where(q, mid, hi2)

    _, cut = jax.lax.fori_loop(0, 12, body2, (z, jnp.full((), t, jnp.int32)))
    return thresh, cut


def _stt_body(x_a_ref, x_b_ref, nw1_ref, wg1_ref, wu1_ref, wd1_ref, nw2_ref,
              wg2_ref, wu2_ref, wd2_ref, betas_ref, out_ref, proc_s,
              shared_s, sc2_s, g_s, th_s, *, d, k, seq, nf, nt, m, mc, nblk,
              tiles_per_seq, bsz):
    s = pl.program_id(0)
    c = pl.program_id(1)
    i = pl.program_id(2)
    base = i * m

    @pl.when((s == 0) & (c == 0))
    def _mlp1_first():
        x = x_a_ref[...]
        v = jnp.mean(x * x, axis=-1, keepdims=True)
        xn = (x * jax.lax.rsqrt(v + _EPS)) * nw1_ref[...]
        shared_s[pl.ds(base, m), :] = xn
        proc_s[pl.ds(base, m), :] = x + _swiglu_chunk(xn, wg1_ref, wu1_ref,
                                                      wd1_ref)

    if nf > 1:
        @pl.when((s == 0) & (c > 0))
        def _mlp1_rest():
            xn = shared_s[pl.ds(base, m), :]
            proc_s[pl.ds(base, m), :] = (
                proc_s[pl.ds(base, m), :]
                + _swiglu_chunk(xn, wg1_ref, wu1_ref, wd1_ref))

    def _prev_tile():
        proc = proc_s[pl.ds(base, m), :]
        last = proc_s[pl.ds(jnp.maximum(base - 1, 0), 1), :]
        last = jnp.where((i % tiles_per_seq) == 0, 0.0, last)
        return jnp.concatenate([last, proc[:m - 1, :]], axis=0)

    def _score_epilogue(p):
        x = x_a_ref[...]
        r = proc_s[pl.ds(base, m), :] - x
        d_st = jnp.sum(r * r, axis=-1, keepdims=True) * (1.0 / d)
        e = r - p
        d_ch = jnp.sum(e * e, axis=-1, keepdims=True) * (1.0 / d)
        g_s[pl.ds(base, m), :] = jax.nn.sigmoid(
            betas_ref[0, 0] * d_st - betas_ref[0, 1] * d_ch)

    @pl.when((s == 1) & (c == 0))
    def _tpn_first():
        prev = _prev_tile()
        v = jnp.mean(prev * prev, axis=-1, keepdims=True)
        scale = jax.lax.rsqrt(v + _EPS)
        sc2_s[pl.ds(base, m), :] = scale
        xn = (prev * scale) * nw2_ref[...]
        t = _swiglu_chunk(xn, wg2_ref, wu2_ref, wd2_ref)
        if nf > 1:
            shared_s[pl.ds(base, m), :] = t
        else:
            _score_epilogue(t)

    if nf > 1:
        @pl.when((s == 1) & (c > 0))
        def _tpn_rest():
            prev = _prev_tile()
            xn = (prev * sc2_s[pl.ds(base, m), :]) * nw2_ref[...]
            t = shared_s[pl.ds(base, m), :] + _swiglu_chunk(
                xn, wg2_ref, wu2_ref, wd2_ref)

            @pl.when(c < nf - 1)
            def _():
                shared_s[pl.ds(base, m), :] = t

            @pl.when(c == nf - 1)
            def _():
                _score_epilogue(t)

    @pl.when((s == 2) & (c == 0) & (i == 0))
    def _thresholds():
        for b in range(bsz):
            grow = g_s[pl.ds(b * seq, seq), 0:1]
            gb = jax.lax.bitcast_convert_type(
                jnp.reshape(grow, (seq // 128, 128)), jnp.int32)
            thresh, cut = _row_threshold(gb, k, seq)
            th_s[pl.ds(b, 1), :] = jnp.broadcast_to(thresh, (1, 128))
            th_s[pl.ds(bsz + b, 1), :] = jnp.broadcast_to(cut, (1, 128))

    @pl.when((s == 2) & (c * nt + i < nblk))
    def _combine():
        flat = c * nt + i
        cbase = flat * mc
        b = cbase // seq
        thr_b = th_s[pl.ds(b, 1), 0:1]
        cut_b = th_s[pl.ds(b + bsz, 1), 0:1]
        g = g_s[pl.ds(cbase, mc), :]
        gb = jax.lax.bitcast_convert_type(g, jnp.int32)
        tloc = jax.lax.broadcasted_iota(jnp.int32, (mc, 1), 0) + (cbase % seq)
        mask = (gb > thr_b) | ((gb == thr_b) & (tloc < cut_b))
        x = x_b_ref[...]
        gated = jnp.where(mask, g, jnp.zeros_like(g))
        out_ref[...] = x + gated * (proc_s[pl.ds(cbase, mc), :] - x)


def kernel(hidden_states, beta_ce, beta_cu, blk_norm_w, blk_wg, blk_wu,
           blk_wd, tpn_norm_w, tpn_wg, tpn_wu, tpn_wd, cr_w, cr_b):
    bsz, seq, d = hidden_states.shape
    f = blk_wg.shape[1]
    n = bsz * seq
    m = min(512, seq)
    nt = n // m
    nf = max(1, f // 256) if f % 256 == 0 else 1
    cf = f // nf
    mc = 64
    while n % mc != 0 or seq % mc != 0 or n // mc > nf * nt:
        mc *= 2
    nblk = n // mc
    k = max(1, int(seq * _CAPACITY))
    cparams = pltpu.CompilerParams(vmem_limit_bytes=60 * 1024 * 1024)

    x = hidden_states.reshape(n, d)
    nw1 = blk_norm_w.reshape(1, d)
    nw2 = tpn_norm_w.reshape(1, d)
    betas = jnp.stack([jnp.asarray(beta_ce, jnp.float32),
                       jnp.asarray(beta_cu, jnp.float32)]).reshape(1, 2)

    const = lambda shape: pl.BlockSpec(shape, lambda s, c, i: (0,) * len(shape))
    out = pl.pallas_call(
        functools.partial(_stt_body, d=float(d), k=k, seq=seq, nf=nf, nt=nt,
                          m=m, mc=mc, nblk=nblk, tiles_per_seq=seq // m,
                          bsz=bsz),
        grid=(3, nf, nt),
        in_specs=[
            pl.BlockSpec((m, d), lambda s, c, i: (
                jnp.where((s == 0) & (c == 0), i,
                          jnp.where((s == 1) & (c == nf - 1), i, nt - 1)),
                0)),
            pl.BlockSpec((mc, d), lambda s, c, i: (
                jnp.where(s == 2, jnp.minimum(c * nt + i, nblk - 1), 0), 0)),
            const((1, d)),
            pl.BlockSpec((d, cf), lambda s, c, i: (
                0, jnp.where(s == 0, c, nf - 1))),
            pl.BlockSpec((d, cf), lambda s, c, i: (
                0, jnp.where(s == 0, c, nf - 1))),
            pl.BlockSpec((cf, d), lambda s, c, i: (
                jnp.where(s == 0, c, nf - 1), 0)),
            const((1, d)),
            pl.BlockSpec((d, cf), lambda s, c, i: (
                0, jnp.where(s == 0, 0, jnp.where(s == 1, c, nf - 1)))),
            pl.BlockSpec((d, cf), lambda s, c, i: (
                0, jnp.where(s == 0, 0, jnp.where(s == 1, c, nf - 1)))),
            pl.BlockSpec((cf, d), lambda s, c, i: (
                jnp.where(s == 0, 0, jnp.where(s == 1, c, nf - 1)), 0)),
            const((1, 2)),
        ],
        out_specs=pl.BlockSpec((mc, d), lambda s, c, i: (
            jnp.where(s == 2, jnp.minimum(c * nt + i, nblk - 1), 0), 0)),
        out_shape=jax.ShapeDtypeStruct((n, d), jnp.float32),
        scratch_shapes=[
            pltpu.VMEM((n, d), jnp.float32),
            pltpu.VMEM((n, d), jnp.float32),
            pltpu.VMEM((n, 1), jnp.float32),
            pltpu.VMEM((n, 1), jnp.float32),
            pltpu.VMEM((2 * bsz, 128), jnp.int32),
        ],
        compiler_params=cparams,
    )(x, x, nw1, blk_wg, blk_wu, blk_wd, nw2, tpn_wg, tpn_wu, tpn_wd, betas)

    return out.reshape(bsz, seq, d)


# speedup vs baseline: 1.2709x; 1.2709x over previous
import functools

import jax
import jax.numpy as jnp
from jax.experimental import pallas as pl
from jax.experimental.pallas import tpu as pltpu

_EPS = 1e-6
_CAPACITY = 0.5


def _rmsnorm(x, w):
    v = jnp.mean(x * x, axis=-1, keepdims=True)
    return (x * jax.lax.rsqrt(v + _EPS)) * w


def _swiglu(xn, wg_ref, wu_ref, wd_ref, nchunks=11):
    f = wg_ref.shape[-1]
    while f % (nchunks * 128) != 0 and nchunks > 1:
        nchunks //= 2
    cf = f // nchunks
    acc = None
    for c in range(nchunks):
        a = jnp.dot(xn, wg_ref[:, c * cf:(c + 1) * cf],
                    preferred_element_type=jnp.float32)
        b = jnp.dot(xn, wu_ref[:, c * cf:(c + 1) * cf],
                    preferred_element_type=jnp.float32)
        h = (a * jax.nn.sigmoid(a)) * b
        t = jnp.dot(h, wd_ref[c * cf:(c + 1) * cf, :],
                    preferred_element_type=jnp.float32)
        acc = t if acc is None else acc + t
    return acc


def _mlp_block_body(x_ref, nw_ref, wg_ref, wu_ref, wd_ref, out_ref):
    x = x_ref[...]
    out_ref[...] = x + _swiglu(_rmsnorm(x, nw_ref[...]), wg_ref, wu_ref,
                               wd_ref)


def _row_threshold(gb, k, t):
    def body1(_, lohi):
        lo, hi = lohi
        mid = lo + (hi - lo + 1) // 2
        cnt = jnp.sum((gb >= mid).astype(jnp.int32))
        pred = cnt >= k
        return jnp.where(pred, mid, lo), jnp.where(pred, hi, mid - 1)

    z = jnp.zeros((), jnp.int32)
    thresh, _ = jax.lax.fori_loop(
        0, 31, body1, (z, jnp.full((), 0x3F800000, jnp.int32)))

    n_gt = jnp.sum((gb > thresh).astype(jnp.int32))
    need = k - n_gt
    eq = gb == thresh
    lanes = gb.shape[-1]
    iota = (jax.lax.broadcasted_iota(jnp.int32, gb.shape, 0) * lanes
            + jax.lax.broadcasted_iota(jnp.int32, gb.shape, 1))

    def body2(_, lohi):
        lo2, hi2 = lohi
        mid = (lo2 + hi2) // 2
        q = jnp.sum((eq & (iota < mid)).astype(jnp.int32)) >= need
        return jnp.where(q, lo2, mid), jnp.where(q, mid, hi2)

    _, cut = jax.lax.fori_loop(0, 12, body2, (z, jnp.full((), t, jnp.int32)))
    return thresh, cut


def _tpn_route_combine_body(proc_ref, x_ref, nw_ref, wg_ref, wu_ref, wd_ref,
                            betas_ref, out_ref, proc_s, g_s, th_s,
                            *, d, k, seq, tiles_per_seq, bsz):
    s = pl.program_id(0)
    i = pl.program_id(1)
    m = x_ref.shape[0]
    base = i * m

    @pl.when(s == 0)
    def _tpn():
        proc = proc_ref[...]
        proc_s[pl.ds(base, m), :] = proc
        last = proc_s[pl.ds(jnp.maximum(base - 1, 0), 1), :]
        last = jnp.where((i % tiles_per_seq) == 0, 0.0, last)
        prev = jnp.concatenate([last, proc[:m - 1, :]], axis=0)
        xn = _rmsnorm(prev, nw_ref[...])
        p = _swiglu(xn, wg_ref, wu_ref, wd_ref)
        r = proc - x_ref[...]
        d_st = jnp.sum(r * r, axis=-1, keepdims=True) * (1.0 / d)
        e = r - p
        d_ch = jnp.sum(e * e, axis=-1, keepdims=True) * (1.0 / d)
        g_s[pl.ds(base, m), :] = jax.nn.sigmoid(
            betas_ref[0, 0] * d_st - betas_ref[0, 1] * d_ch)

    @pl.when((s == 1) & (i == 0))
    def _thresholds():
        for b in range(bsz):
            grow = g_s[pl.ds(b * seq, seq), 0:1]
            gb = jax.lax.bitcast_convert_type(
                jnp.reshape(grow, (seq // 128, 128)), jnp.int32)
            thresh, cut = _row_threshold(gb, k, seq)
            th_s[pl.ds(b, 1), :] = jnp.broadcast_to(thresh, (1, 128))
            th_s[pl.ds(bsz + b, 1), :] = jnp.broadcast_to(cut, (1, 128))

    @pl.when(s == 1)
    def _combine():
        b = i // tiles_per_seq
        t0 = (i % tiles_per_seq) * m
        thr_b = th_s[pl.ds(b, 1), 0:1]
        cut_b = th_s[pl.ds(b + bsz, 1), 0:1]
        g = g_s[pl.ds(base, m), :]
        gb = jax.lax.bitcast_convert_type(g, jnp.int32)
        tloc = jax.lax.broadcasted_iota(jnp.int32, (m, 1), 0) + t0
        mask = (gb > thr_b) | ((gb == thr_b) & (tloc < cut_b))
        x = x_ref[...]
        gated = jnp.where(mask, g, jnp.zeros_like(g))
        out_ref[...] = x + gated * (proc_s[pl.ds(base, m), :] - x)


def kernel(hidden_states, beta_ce, beta_cu, blk_norm_w, blk_wg, blk_wu,
           blk_wd, tpn_norm_w, tpn_wg, tpn_wu, tpn_wd, cr_w, cr_b):
    bsz, seq, d = hidden_states.shape
    f = blk_wg.shape[1]
    n = bsz * seq
    m1 = min(1024, seq)
    m2 = min(256, seq)
    nt2 = n // m2
    cparams = pltpu.CompilerParams(vmem_limit_bytes=63 * 1024 * 1024)
    k = max(1, int(seq * _CAPACITY))

    x = hidden_states.reshape(n, d)
    nw1 = blk_norm_w.reshape(1, d)
    nw2 = tpn_norm_w.reshape(1, d)

    processed = pl.pallas_call(
        _mlp_block_body,
        grid=(n // m1,),
        in_specs=[pl.BlockSpec((m1, d), lambda i: (i, 0)),
                  pl.BlockSpec((1, d), lambda i: (0, 0)),
                  pl.BlockSpec((d, f), lambda i: (0, 0)),
                  pl.BlockSpec((d, f), lambda i: (0, 0)),
                  pl.BlockSpec((f, d), lambda i: (0, 0))],
        out_specs=pl.BlockSpec((m1, d), lambda i: (i, 0)),
        out_shape=jax.ShapeDtypeStruct((n, d), jnp.float32),
        compiler_params=cparams,
    )(x, nw1, blk_wg, blk_wu, blk_wd)

    betas = jnp.stack([jnp.asarray(beta_ce, jnp.float32),
                       jnp.asarray(beta_cu, jnp.float32)]).reshape(1, 2)

    const2 = lambda shape: pl.BlockSpec(shape, lambda s, i: (0,) * len(shape))
    out = pl.pallas_call(
        functools.partial(_tpn_route_combine_body, d=float(d), k=k, seq=seq,
                          tiles_per_seq=seq // m2, bsz=bsz),
        grid=(2, nt2),
        in_specs=[pl.BlockSpec((m2, d),
                               lambda s, i: (jnp.where(s == 0, i, nt2 - 1),
                                             0)),
                  pl.BlockSpec((m2, d), lambda s, i: (i, 0)),
                  const2((1, d)), const2((d, f)), const2((d, f)),
                  const2((f, d)), const2((1, 2))],
        out_specs=pl.BlockSpec((m2, d),
                               lambda s, i: (jnp.where(s == 1, i, 0), 0)),
        out_shape=jax.ShapeDtypeStruct((n, d), jnp.float32),
        scratch_shapes=[
            pltpu.VMEM((n, d), jnp.float32),
            pltpu.VMEM((n, 1), jnp.float32),
            pltpu.VMEM((2 * bsz, 128), jnp.int32),
        ],
        compiler_params=cparams,
    )(processed, x, nw2, tpn_wg, tpn_wu, tpn_wd, betas)

    return out.reshape(bsz, seq, d)
